# baseline (speedup 1.0000x reference)
import functools

import jax
import jax.numpy as jnp
from jax import lax
from jax.experimental import pallas as pl
from jax.experimental.pallas import tpu as pltpu
from jax.experimental.pallas import tpu_sc as plsc

VOCAB = 1000000
HID = 64
MAXPOS = 512
TYPES = 2
N_TOK = 1024 * 200
NW = 32
CH = 128
CHUNKS = N_TOK // (NW * CH)
G_PER_CH = CH // 16


def _rsqrt(x):
    i = lax.bitcast_convert_type(x, jnp.int32)
    i = jnp.int32(0x5F3759DF) - lax.shift_right_logical(i, 1)
    y = lax.bitcast_convert_type(i, jnp.float32)
    for _ in range(3):
        y = y * (1.5 - 0.5 * x * y * y)
    return y


def _sc_body(tok_ids_h, pos_ids_h, typ_ids_h, token_table_h, pos_table_h,
             typ_table_h, gamma_h, beta_h, out_h,
             tok_ids_v, pos_ids_v, typ_ids_v, rows_v, out_v,
             pos_v, typ_v, gam_v, bet_v, emb_s, gs_s, bs_s, sem):
    wid = lax.axis_index("s") * 2 + lax.axis_index("c")

    pltpu.sync_copy(pos_table_h, pos_v)
    pltpu.sync_copy(typ_table_h, typ_v)
    pltpu.sync_copy(gamma_h, gam_v)
    pltpu.sync_copy(beta_h, bet_v)

    for k in range(HID // 16):
        gv = gam_v[pl.ds(k * 16, 16)]
        bv = bet_v[pl.ds(k * 16, 16)]
        for l in range(16):
            gs_s[k * 16 + l] = jnp.broadcast_to(gv[l], (16,))
            bs_s[k * 16 + l] = jnp.broadcast_to(bv[l], (16,))

    iota16 = lax.iota(jnp.int32, 16)

    def chunk_body(c, carry):
        row = wid * CHUNKS + c
        pltpu.sync_copy(tok_ids_h.at[row], tok_ids_v)
        pltpu.sync_copy(pos_ids_h.at[row], pos_ids_v)
        pltpu.sync_copy(typ_ids_h.at[row], typ_ids_v)
        pltpu.async_copy(token_table_h.at[tok_ids_v], rows_v, sem).wait()

        def group_body(g, carry2):
            base = g * 16
            ridx = iota16 + base
            pids = plsc.load_gather(pos_ids_v, [ridx])
            tids = plsc.load_gather(typ_ids_v, [ridx])

            s = jnp.zeros((16,), jnp.float32)
            q = jnp.zeros((16,), jnp.float32)
            for j in range(HID):
                cj = jnp.full((16,), j, jnp.int32)
                t = plsc.load_gather(rows_v, [ridx, cj])
                p = plsc.load_gather(pos_v, [pids, cj])
                y = plsc.load_gather(typ_v, [tids, cj])
                e = t + p + y
                s = s + e
                q = q + e * e
                emb_s[j] = e

            mean = s * (1.0 / HID)
            var = q * (1.0 / HID) - mean * mean
            r = _rsqrt(var + 1e-12)
            bias = -mean * r

            for j in range(HID):
                e = emb_s[j]
                ns = e * r + bias
                plsc.store_scatter(out_v, [ridx, jnp.full((16,), j, jnp.int32)],
                                   ns * gs_s[j] + bs_s[j])
            return carry2

        lax.fori_loop(0, G_PER_CH, group_body, 0)
        pltpu.sync_copy(out_v, out_h.at[row])
        return carry

    lax.fori_loop(0, CHUNKS, chunk_body, 0)


@jax.jit
def _run(tok_ids, pos_ids, typ_ids, token_table, pos_table, typ_table,
         gamma, beta):
    mesh = plsc.VectorSubcoreMesh(core_axis_name="c", subcore_axis_name="s")
    kern = pl.kernel(
        _sc_body,
        out_type=jax.ShapeDtypeStruct((NW * CHUNKS, CH, HID), jnp.float32),
        mesh=mesh,
        compiler_params=pltpu.CompilerParams(
            needs_layout_passes=False, use_tc_tiling_on_sc=False),
        scratch_types=[
            pltpu.VMEM((CH,), jnp.int32),
            pltpu.VMEM((CH,), jnp.int32),
            pltpu.VMEM((CH,), jnp.int32),
            pltpu.VMEM((CH, HID), jnp.float32),
            pltpu.VMEM((CH, HID), jnp.float32),
            pltpu.VMEM((MAXPOS, HID), jnp.float32),
            pltpu.VMEM((TYPES, HID), jnp.float32),
            pltpu.VMEM((HID,), jnp.float32),
            pltpu.VMEM((HID,), jnp.float32),
            pltpu.VMEM((HID, 16), jnp.float32),
            pltpu.VMEM((HID, 16), jnp.float32),
            pltpu.VMEM((HID, 16), jnp.float32),
            pltpu.SemaphoreType.DMA,
        ],
    )
    return kern(tok_ids, pos_ids, typ_ids, token_table, pos_table, typ_table,
                gamma, beta)


def kernel(input_ids, position_ids, token_type_ids, token_table,
           position_table, type_table, gamma, beta):
    B, S = input_ids.shape
    rows = NW * CHUNKS
    tok_ids = input_ids.reshape(rows, CH)
    pos_ids = position_ids.reshape(rows, CH)
    typ_ids = token_type_ids.reshape(rows, CH)
    out = _run(tok_ids, pos_ids, typ_ids, token_table, position_table,
               type_table, gamma, beta)
    return out.reshape(B, S, HID)


# speedup vs baseline: 1.0518x; 1.0518x over previous
import jax
import jax.numpy as jnp
from jax import lax
from jax.experimental import pallas as pl
from jax.experimental.pallas import tpu as pltpu
from jax.experimental.pallas import tpu_sc as plsc

VOCAB = 1000000
HID = 64
MAXPOS = 512
TYPES = 2
N_TOK = 1024 * 200
NW = 32
CH = 128
CHUNKS = N_TOK // (NW * CH)
G_PER_CH = CH // 16


def _rsqrt(x):
    i = lax.bitcast_convert_type(x, jnp.int32)
    i = jnp.int32(0x5F3759DF) - lax.shift_right_logical(i, 1)
    y = lax.bitcast_convert_type(i, jnp.float32)
    for _ in range(3):
        y = y * (1.5 - 0.5 * x * y * y)
    return y


def _sc_body(ids_h, token_table_h, pos_table_h, typ_table_h, gamma_h, beta_h,
             out_h,
             ids0, ids1, rows0, rows1, outb0, outb1,
             pos_v, typ_v, gam_v, bet_v, emb_s, gs_s, bs_s,
             gsem0, gsem1, osem0, osem1):
    wid = lax.axis_index("s") * 2 + lax.axis_index("c")
    base_row = wid * CHUNKS
    ids_b = (ids0, ids1)
    rows_b = (rows0, rows1)
    out_b = (outb0, outb1)
    gsem = (gsem0, gsem1)
    osem = (osem0, osem1)

    pltpu.sync_copy(pos_table_h, pos_v)
    pltpu.sync_copy(typ_table_h, typ_v)
    pltpu.sync_copy(gamma_h, gam_v)
    pltpu.sync_copy(beta_h, bet_v)

    for k in range(HID // 16):
        gv = gam_v[pl.ds(k * 16, 16)]
        bv = bet_v[pl.ds(k * 16, 16)]
        for l in range(16):
            gs_s[k * 16 + l] = jnp.broadcast_to(gv[l], (16,))
            bs_s[k * 16 + l] = jnp.broadcast_to(bv[l], (16,))

    iota16 = lax.iota(jnp.int32, 16)

    def fetch(c, par):
        pltpu.sync_copy(ids_h.at[base_row + c], ids_b[par])
        pltpu.async_copy(token_table_h.at[ids_b[par].at[0]], rows_b[par],
                         gsem[par])

    def compute(c, par):
        rows_v = rows_b[par]
        ids_v = ids_b[par]
        out_v = out_b[par]

        def group_body(g, carry):
            base = g * 16
            ridx = iota16 + base
            pids = plsc.load_gather(ids_v, [jnp.full((16,), 1, jnp.int32), ridx])
            tids = plsc.load_gather(ids_v, [jnp.full((16,), 2, jnp.int32), ridx])

            s = jnp.zeros((16,), jnp.float32)
            q = jnp.zeros((16,), jnp.float32)
            for j in range(HID):
                cj = jnp.full((16,), j, jnp.int32)
                t = plsc.load_gather(rows_v, [ridx, cj])
                p = plsc.load_gather(pos_v, [pids, cj])
                y = plsc.load_gather(typ_v, [tids, cj])
                e = t + p + y
                s = s + e
                q = q + e * e
                emb_s[j] = e

            mean = s * (1.0 / HID)
            var = q * (1.0 / HID) - mean * mean
            r = _rsqrt(var + 1e-12)
            bias = -mean * r

            for j in range(HID):
                e = emb_s[j]
                ns = e * r + bias
                plsc.store_scatter(out_v, [ridx, jnp.full((16,), j, jnp.int32)],
                                   ns * gs_s[j] + bs_s[j])
            return carry

        lax.fori_loop(0, G_PER_CH, group_body, 0)

    fetch(0, 0)
    fetch(1, 1)

    def pair_body(cc, carry):
        for par in range(2):
            c = cc * 2 + par
            pltpu.make_async_copy(token_table_h.at[ids_b[par].at[0]],
                                  rows_b[par], gsem[par]).wait()

            @pl.when(c >= 2)
            def _():
                pltpu.make_async_copy(out_b[par], out_h.at[base_row + c - 2],
                                      osem[par]).wait()

            compute(c, par)
            pltpu.async_copy(out_b[par], out_h.at[base_row + c], osem[par])

            @pl.when(c + 2 < CHUNKS)
            def _():
                fetch(c + 2, par)
        return carry

    lax.fori_loop(0, CHUNKS // 2, pair_body, 0)

    pltpu.make_async_copy(out_b[0], out_h.at[base_row + CHUNKS - 2],
                          osem[0]).wait()
    pltpu.make_async_copy(out_b[1], out_h.at[base_row + CHUNKS - 1],
                          osem[1]).wait()


@jax.jit
def _run(ids, token_table, pos_table, typ_table, gamma, beta):
    mesh = plsc.VectorSubcoreMesh(core_axis_name="c", subcore_axis_name="s")
    kern = pl.kernel(
        _sc_body,
        out_type=jax.ShapeDtypeStruct((NW * CHUNKS, CH, HID), jnp.float32),
        mesh=mesh,
        compiler_params=pltpu.CompilerParams(
            needs_layout_passes=False, use_tc_tiling_on_sc=False),
        scratch_types=[
            pltpu.VMEM((3, CH), jnp.int32),
            pltpu.VMEM((3, CH), jnp.int32),
            pltpu.VMEM((CH, HID), jnp.float32),
            pltpu.VMEM((CH, HID), jnp.float32),
            pltpu.VMEM((CH, HID), jnp.float32),
            pltpu.VMEM((CH, HID), jnp.float32),
            pltpu.VMEM((MAXPOS, HID), jnp.float32),
            pltpu.VMEM((TYPES, HID), jnp.float32),
            pltpu.VMEM((HID,), jnp.float32),
            pltpu.VMEM((HID,), jnp.float32),
            pltpu.VMEM((HID, 16), jnp.float32),
            pltpu.VMEM((HID, 16), jnp.float32),
            pltpu.VMEM((HID, 16), jnp.float32),
            pltpu.SemaphoreType.DMA,
            pltpu.SemaphoreType.DMA,
            pltpu.SemaphoreType.DMA,
            pltpu.SemaphoreType.DMA,
        ],
    )
    return kern(ids, token_table, pos_table, typ_table, gamma, beta)


def kernel(input_ids, position_ids, token_type_ids, token_table,
           position_table, type_table, gamma, beta):
    B, S = input_ids.shape
    rows = NW * CHUNKS
    ids = jnp.stack([input_ids.reshape(rows, CH),
                     position_ids.reshape(rows, CH),
                     token_type_ids.reshape(rows, CH)], axis=1)
    out = _run(ids, token_table, position_table, type_table, gamma, beta)
    return out.reshape(B, S, HID)


# speedup vs baseline: 1.8607x; 1.7691x over previous
import jax
import jax.numpy as jnp
from jax import lax
from jax.experimental import pallas as pl
from jax.experimental.pallas import tpu as pltpu
from jax.experimental.pallas import tpu_sc as plsc

VOCAB = 1000000
HID = 64
NK = HID // 16
TYPES = 2
N_TOK = 1024 * 200
NW = 32
CH = 128
CHUNKS = N_TOK // (NW * CH)
G_PER_CH = CH // 16


def _rsqrt_scalar(x):
    i = lax.bitcast_convert_type(x, jnp.int32)
    i = jnp.int32(0x5F3759DF) - lax.shift_right_logical(i, 1)
    y = lax.bitcast_convert_type(i, jnp.float32)
    for _ in range(3):
        y = y * (1.5 - 0.5 * x * y * y)
    return y


def _sc_body(ids_h, token_table_h, pos_table_h, typ_table_h, gamma_h, beta_h,
             out_h,
             ids0, ids1, rows0, rows1, prows0, prows1, outb0, outb1,
             typ_v, gam_v, bet_v,
             gsem0, gsem1, psem0, psem1, osem0, osem1):
    wid = lax.axis_index("s") * 2 + lax.axis_index("c")
    base_row = wid * CHUNKS
    ids_b = (ids0, ids1)
    rows_b = (rows0, rows1)
    prows_b = (prows0, prows1)
    out_b = (outb0, outb1)
    gsem = (gsem0, gsem1)
    psem = (psem0, psem1)
    osem = (osem0, osem1)

    pltpu.sync_copy(typ_table_h, typ_v)
    pltpu.sync_copy(gamma_h, gam_v)
    pltpu.sync_copy(beta_h, bet_v)

    t0k = [typ_v[0, pl.ds(k * 16, 16)] for k in range(NK)]
    t1k = [typ_v[1, pl.ds(k * 16, 16)] for k in range(NK)]
    gk = [gam_v[pl.ds(k * 16, 16)] for k in range(NK)]
    bk = [bet_v[pl.ds(k * 16, 16)] for k in range(NK)]

    iota16 = lax.iota(jnp.int32, 16)

    def fetch(c, par):
        pltpu.sync_copy(ids_h.at[base_row + c], ids_b[par])
        pltpu.async_copy(token_table_h.at[ids_b[par].at[0]], rows_b[par],
                         gsem[par])
        pltpu.async_copy(pos_table_h.at[ids_b[par].at[1]], prows_b[par],
                         psem[par])

    def compute(par):
        rows_v = rows_b[par]
        prow_v = prows_b[par]
        ids_v = ids_b[par]
        out_v = out_b[par]

        def group_body(g, carry):
            base = g * 16
            tids = plsc.load_gather(
                ids_v, [jnp.full((16,), 2, jnp.int32), iota16 + base])
            for t in range(16):
                row = base + t
                is0 = tids[t] == 0
                e = []
                for k in range(NK):
                    ty = jnp.where(is0, t0k[k], t1k[k])
                    e.append(rows_v[row, pl.ds(k * 16, 16)]
                             + prow_v[row, pl.ds(k * 16, 16)] + ty)
                s = (e[0] + e[1]) + (e[2] + e[3])
                q = (e[0] * e[0] + e[1] * e[1]) + (e[2] * e[2] + e[3] * e[3])
                ssum = jnp.sum(s)
                qsum = jnp.sum(q)
                mean = ssum * (1.0 / HID)
                var = qsum * (1.0 / HID) - mean * mean
                r = _rsqrt_scalar(var + 1e-12)
                b2 = -mean * r
                rv = jnp.full((16,), r, jnp.float32)
                bv = jnp.full((16,), b2, jnp.float32)
                for k in range(NK):
                    ns = e[k] * rv + bv
                    out_v[row, pl.ds(k * 16, 16)] = ns * gk[k] + bk[k]
            return carry

        lax.fori_loop(0, G_PER_CH, group_body, 0)

    fetch(0, 0)
    fetch(1, 1)

    def pair_body(cc, carry):
        for par in range(2):
            c = cc * 2 + par
            pltpu.make_async_copy(token_table_h.at[ids_b[par].at[0]],
                                  rows_b[par], gsem[par]).wait()
            pltpu.make_async_copy(pos_table_h.at[ids_b[par].at[1]],
                                  prows_b[par], psem[par]).wait()

            @pl.when(c >= 2)
            def _():
                pltpu.make_async_copy(out_b[par], out_h.at[base_row + c - 2],
                                      osem[par]).wait()

            compute(par)
            pltpu.async_copy(out_b[par], out_h.at[base_row + c], osem[par])

            @pl.when(c + 2 < CHUNKS)
            def _():
                fetch(c + 2, par)
        return carry

    lax.fori_loop(0, CHUNKS // 2, pair_body, 0)

    pltpu.make_async_copy(out_b[0], out_h.at[base_row + CHUNKS - 2],
                          osem[0]).wait()
    pltpu.make_async_copy(out_b[1], out_h.at[base_row + CHUNKS - 1],
                          osem[1]).wait()


@jax.jit
def _run(ids, token_table, pos_table, typ_table, gamma, beta):
    mesh = plsc.VectorSubcoreMesh(core_axis_name="c", subcore_axis_name="s")
    kern = pl.kernel(
        _sc_body,
        out_type=jax.ShapeDtypeStruct((NW * CHUNKS, CH, HID), jnp.float32),
        mesh=mesh,
        compiler_params=pltpu.CompilerParams(
            needs_layout_passes=False, use_tc_tiling_on_sc=False),
        scratch_types=[
            pltpu.VMEM((3, CH), jnp.int32),
            pltpu.VMEM((3, CH), jnp.int32),
            pltpu.VMEM((CH, HID), jnp.float32),
            pltpu.VMEM((CH, HID), jnp.float32),
            pltpu.VMEM((CH, HID), jnp.float32),
            pltpu.VMEM((CH, HID), jnp.float32),
            pltpu.VMEM((CH, HID), jnp.float32),
            pltpu.VMEM((CH, HID), jnp.float32),
            pltpu.VMEM((TYPES, HID), jnp.float32),
            pltpu.VMEM((HID,), jnp.float32),
            pltpu.VMEM((HID,), jnp.float32),
            pltpu.SemaphoreType.DMA,
            pltpu.SemaphoreType.DMA,
            pltpu.SemaphoreType.DMA,
            pltpu.SemaphoreType.DMA,
            pltpu.SemaphoreType.DMA,
            pltpu.SemaphoreType.DMA,
        ],
    )
    return kern(ids, token_table, pos_table, typ_table, gamma, beta)


def kernel(input_ids, position_ids, token_type_ids, token_table,
           position_table, type_table, gamma, beta):
    B, S = input_ids.shape
    rows = NW * CHUNKS
    ids = jnp.stack([input_ids.reshape(rows, CH),
                     position_ids.reshape(rows, CH),
                     token_type_ids.reshape(rows, CH)], axis=1)
    out = _run(ids, token_table, position_table, type_table, gamma, beta)
    return out.reshape(B, S, HID)


# speedup vs baseline: 2.0641x; 1.1093x over previous
import jax
import jax.numpy as jnp
from jax import lax
from jax.experimental import pallas as pl
from jax.experimental.pallas import tpu as pltpu
from jax.experimental.pallas import tpu_sc as plsc

VOCAB = 1000000
HID = 64
NK = HID // 16
TYPES = 2
B = 1024
S = 200
NW = 32
ROWS_PW = B // NW
NG = S // 16
TAIL = S - NG * 16


def _rsqrt_vec(x):
    i = lax.bitcast_convert_type(x, jnp.int32)
    i = jnp.full((16,), 0x5F3759DF, jnp.int32) - lax.shift_right_logical(i, 1)
    y = lax.bitcast_convert_type(i, jnp.float32)
    for _ in range(3):
        y = y * (1.5 - 0.5 * x * y * y)
    return y


def _sc_body(tok_ids_h, pos_ids_h, typ_ids_h, token_table_h, pos_table_h,
             typ_table_h, gamma_h, beta_h, out_h,
             tids0, tids1, pids0, pids1, yids0, yids1,
             rows0, rows1, prows0, prows1, outb0, outb1,
             typ_v, gam_v, bet_v, stat_s,
             gsem0, gsem1, psem0, psem1, osem0, osem1):
    wid = lax.axis_index("s") * 2 + lax.axis_index("c")
    brow = wid * ROWS_PW
    tids_b = (tids0, tids1)
    pids_b = (pids0, pids1)
    yids_b = (yids0, yids1)
    rows_b = (rows0, rows1)
    prows_b = (prows0, prows1)
    out_b = (outb0, outb1)
    gsem = (gsem0, gsem1)
    psem = (psem0, psem1)
    osem = (osem0, osem1)

    pltpu.sync_copy(typ_table_h, typ_v)
    pltpu.sync_copy(gamma_h, gam_v)
    pltpu.sync_copy(beta_h, bet_v)

    t0k = [typ_v[0, pl.ds(k * 16, 16)] for k in range(NK)]
    t1k = [typ_v[1, pl.ds(k * 16, 16)] for k in range(NK)]
    gk = [gam_v[pl.ds(k * 16, 16)] for k in range(NK)]
    bk = [bet_v[pl.ds(k * 16, 16)] for k in range(NK)]

    lane15 = lax.iota(jnp.int32, 16) == 15

    def fetch(c, par):
        pltpu.sync_copy(tok_ids_h.at[brow + c], tids_b[par])
        pltpu.sync_copy(pos_ids_h.at[brow + c], pids_b[par])
        pltpu.sync_copy(typ_ids_h.at[brow + c], yids_b[par])
        pltpu.async_copy(token_table_h.at[tids_b[par].at[pl.ds(0, 128)]],
                         rows_b[par].at[pl.ds(0, 128)], gsem[par])
        pltpu.async_copy(token_table_h.at[tids_b[par].at[pl.ds(128, 72)]],
                         rows_b[par].at[pl.ds(128, 72)], gsem[par])
        pltpu.async_copy(pos_table_h.at[pids_b[par].at[pl.ds(0, 128)]],
                         prows_b[par].at[pl.ds(0, 128)], psem[par])
        pltpu.async_copy(pos_table_h.at[pids_b[par].at[pl.ds(128, 72)]],
                         prows_b[par].at[pl.ds(128, 72)], psem[par])

    def wait_fetch(par):
        pltpu.make_async_copy(token_table_h.at[tids_b[par].at[pl.ds(0, 128)]],
                              rows_b[par].at[pl.ds(0, 128)], gsem[par]).wait()
        pltpu.make_async_copy(token_table_h.at[tids_b[par].at[pl.ds(128, 72)]],
                              rows_b[par].at[pl.ds(128, 72)], gsem[par]).wait()
        pltpu.make_async_copy(pos_table_h.at[pids_b[par].at[pl.ds(0, 128)]],
                              prows_b[par].at[pl.ds(0, 128)], psem[par]).wait()
        pltpu.make_async_copy(pos_table_h.at[pids_b[par].at[pl.ds(128, 72)]],
                              prows_b[par].at[pl.ds(128, 72)], psem[par]).wait()

    def do_group(base, n_tok, rows_v, prow_v, yids_v, out_v):
        for t in range(n_tok):
            row = base + t
            tp = jnp.full((16,), row, jnp.int32)
            tid = plsc.load_gather(yids_v, [tp])
            is0 = tid == 0
            e = []
            for k in range(NK):
                ty = jnp.where(is0, t0k[k], t1k[k])
                ek = (rows_v[row, pl.ds(k * 16, 16)]
                      + prow_v[row, pl.ds(k * 16, 16)] + ty)
                e.append(ek)
                out_v[row, pl.ds(k * 16, 16)] = ek
            s = (e[0] + e[1]) + (e[2] + e[3])
            q = (e[0] * e[0] + e[1] * e[1]) + (e[2] * e[2] + e[3] * e[3])
            cs = plsc.cumsum(s)
            cq = plsc.cumsum(q)
            tl = jnp.full((16,), t, jnp.int32)
            plsc.store_scatter(stat_s, [tl], cs, mask=lane15)
            plsc.store_scatter(stat_s, [tl + 16], cq, mask=lane15)

        s_vec = stat_s[pl.ds(0, 16)]
        q_vec = stat_s[pl.ds(16, 16)]
        mean = s_vec * (1.0 / HID)
        var = q_vec * (1.0 / HID) - mean * mean
        r = _rsqrt_vec(var + 1e-12)
        b2 = -mean * r
        stat_s[pl.ds(0, 16)] = r
        stat_s[pl.ds(16, 16)] = b2

        for t in range(n_tok):
            row = base + t
            tl = jnp.full((16,), t, jnp.int32)
            rs = plsc.load_gather(stat_s, [tl])
            bs = plsc.load_gather(stat_s, [tl + 16])
            for k in range(NK):
                ek = out_v[row, pl.ds(k * 16, 16)]
                ns = ek * rs + bs
                out_v[row, pl.ds(k * 16, 16)] = ns * gk[k] + bk[k]

    def compute(par):
        rows_v = rows_b[par]
        prow_v = prows_b[par]
        yids_v = yids_b[par]
        out_v = out_b[par]

        def group_body(g, carry):
            do_group(g * 16, 16, rows_v, prow_v, yids_v, out_v)
            return carry

        lax.fori_loop(0, NG, group_body, 0)
        do_group(NG * 16, TAIL, rows_v, prow_v, yids_v, out_v)

    fetch(0, 0)
    fetch(1, 1)

    def pair_body(cc, carry):
        for par in range(2):
            c = cc * 2 + par
            wait_fetch(par)

            @pl.when(c >= 2)
            def _():
                pltpu.make_async_copy(out_b[par], out_h.at[brow + c - 2],
                                      osem[par]).wait()

            compute(par)
            pltpu.async_copy(out_b[par], out_h.at[brow + c], osem[par])

            @pl.when(c + 2 < ROWS_PW)
            def _():
                fetch(c + 2, par)
        return carry

    lax.fori_loop(0, ROWS_PW // 2, pair_body, 0)

    pltpu.make_async_copy(out_b[0], out_h.at[brow + ROWS_PW - 2],
                          osem[0]).wait()
    pltpu.make_async_copy(out_b[1], out_h.at[brow + ROWS_PW - 1],
                          osem[1]).wait()


@jax.jit
def _run(tok_ids, pos_ids, typ_ids, token_table, pos_table, typ_table,
         gamma, beta):
    mesh = plsc.VectorSubcoreMesh(core_axis_name="c", subcore_axis_name="s")
    kern = pl.kernel(
        _sc_body,
        out_type=jax.ShapeDtypeStruct((B, S, HID), jnp.float32),
        mesh=mesh,
        compiler_params=pltpu.CompilerParams(
            needs_layout_passes=False, use_tc_tiling_on_sc=False),
        scratch_types=[
            pltpu.VMEM((S,), jnp.int32),
            pltpu.VMEM((S,), jnp.int32),
            pltpu.VMEM((S,), jnp.int32),
            pltpu.VMEM((S,), jnp.int32),
            pltpu.VMEM((S,), jnp.int32),
            pltpu.VMEM((S,), jnp.int32),
            pltpu.VMEM((S, HID), jnp.float32),
            pltpu.VMEM((S, HID), jnp.float32),
            pltpu.VMEM((S, HID), jnp.float32),
            pltpu.VMEM((S, HID), jnp.float32),
            pltpu.VMEM((S, HID), jnp.float32),
            pltpu.VMEM((S, HID), jnp.float32),
            pltpu.VMEM((TYPES, HID), jnp.float32),
            pltpu.VMEM((HID,), jnp.float32),
            pltpu.VMEM((HID,), jnp.float32),
            pltpu.VMEM((32,), jnp.float32),
            pltpu.SemaphoreType.DMA,
            pltpu.SemaphoreType.DMA,
            pltpu.SemaphoreType.DMA,
            pltpu.SemaphoreType.DMA,
            pltpu.SemaphoreType.DMA,
            pltpu.SemaphoreType.DMA,
        ],
    )
    return kern(tok_ids, pos_ids, typ_ids, token_table, pos_table, typ_table,
                gamma, beta)


def kernel(input_ids, position_ids, token_type_ids, token_table,
           position_table, type_table, gamma, beta):
    return _run(input_ids, position_ids, token_type_ids, token_table,
                position_table, type_table, gamma, beta)
